# Initial kernel scaffold; baseline (speedup 1.0000x reference)
import jax
import jax.numpy as jnp
from jax.experimental import pallas as pl


def kernel(h0, h1, edge_index, W_fcdst, W_attn):
    raise NotImplementedError("write your pallas kernel here")


# speedup vs baseline: 11.6463x; 11.6463x over previous
import jax
import jax.numpy as jnp
from jax import lax
from jax.experimental import pallas as pl
from jax.experimental.pallas import tpu as pltpu
from jax.experimental.pallas import tpu_sc as plsc

N = 10000
D = 128
E = 320000
NC, NS, L = 2, 16, 16
NW = NC * NS
EB = 128
NB = 79
EPT = NB * EB
E_PAD = NW * EPT
N_PAD = 10112
RPT = N_PAD // NS

_MESH = plsc.VectorSubcoreMesh(
    core_axis_name="c", subcore_axis_name="s", num_cores=NC, num_subcores=NS
)


def _tc_prep_body(h0_ref, h1_ref, wt_ref, was_ref, wad_ref, z_ref, ss_ref, sd_ref):
    z = jnp.dot(h0_ref[...], wt_ref[...], preferred_element_type=jnp.float32)
    z_ref[...] = z
    ss_ref[...] = jnp.dot(h1_ref[...], was_ref[...], preferred_element_type=jnp.float32)
    sd_ref[...] = jnp.dot(z, wad_ref[...], preferred_element_type=jnp.float32)


def _tc_prep(h0, h1, wt, wa_s, wa_d):
    return pl.pallas_call(
        _tc_prep_body,
        out_shape=(
            jax.ShapeDtypeStruct((N, D), jnp.float32),
            jax.ShapeDtypeStruct((N, 1), jnp.float32),
            jax.ShapeDtypeStruct((N, 1), jnp.float32),
        ),
    )(h0, h1, wt, wa_s, wa_d)


def _xlane_max(v, buf):
    for sh in (8, 4, 2, 1):
        buf[pl.ds(0, L)] = v
        idx = lax.broadcasted_iota(jnp.int32, (L,), 0) ^ sh
        v = jnp.maximum(v, plsc.load_gather(buf, [idx]))
    return v


def _p1_body(ss_hbm, sd_hbm, src_hbm, dst_hbm,
             denom_hbm, deg_hbm, ex_hbm,
             ssrc_v, sdst_v, src_v, dst_v, ex_v, db_v, ones_v, zero_v, mred_v,
             denom_sh, deg_sh):
    c = lax.axis_index("c")
    s = lax.axis_index("s")
    wid = c * NS + s
    base_e = wid * EPT

    pltpu.sync_copy(ss_hbm, ssrc_v)
    pltpu.sync_copy(sd_hbm, sdst_v)
    pltpu.sync_copy(src_hbm.at[pl.ds(base_e, EPT)], src_v)
    pltpu.sync_copy(dst_hbm.at[pl.ds(base_e, EPT)], dst_v)

    def _zloop(i, carry):
        zero_v[pl.ds(i * L, L)] = jnp.zeros((L,), jnp.float32)
        return carry
    lax.fori_loop(0, RPT // L, _zloop, 0)
    for i in range(EB // L):
        ones_v[pl.ds(i * L, L)] = jnp.ones((L,), jnp.float32)

    pltpu.sync_copy(zero_v, denom_sh.at[pl.ds(s * RPT, RPT)])
    pltpu.sync_copy(zero_v, deg_sh.at[pl.ds(s * RPT, RPT)])
    plsc.subcore_barrier()

    def _mloop(i, carry):
        ms, md = carry
        sl = pl.ds(i * L, L)
        return jnp.maximum(ms, ssrc_v[sl]), jnp.maximum(md, sdst_v[sl])
    init = jnp.full((L,), -3e38, jnp.float32)
    ms, md = lax.fori_loop(0, N_PAD // L, _mloop, (init, init))
    m = _xlane_max(ms, mred_v) + _xlane_max(md, mred_v)
    mv = jnp.where(m > 0, m, m * 0.01)

    def _eloop(b, carry):
        sl = pl.ds(b * L, L)
        vs = plsc.load_gather(ssrc_v, [src_v[sl]])
        vd = plsc.load_gather(sdst_v, [dst_v[sl]])
        a = vs + vd
        e = jnp.where(a > 0, a, a * 0.01)
        ex_v[sl] = jnp.exp(e - mv)
        return carry
    lax.fori_loop(0, EPT // L, _eloop, 0)

    def _sloop(j, carry):
        for i in range(EB // L):
            db_v[pl.ds(i * L, L)] = dst_v[pl.ds(j * EB + i * L, L)]
        pltpu.sync_copy(ex_v.at[pl.ds(j * EB, EB)], denom_sh.at[db_v], add=True)
        pltpu.sync_copy(ones_v, deg_sh.at[db_v], add=True)
        return carry
    lax.fori_loop(0, NB, _sloop, 0)
    plsc.subcore_barrier()

    off_n = c * N_PAD + s * RPT
    pltpu.sync_copy(denom_sh.at[pl.ds(s * RPT, RPT)], zero_v)
    pltpu.sync_copy(zero_v, denom_hbm.at[pl.ds(off_n, RPT)])
    pltpu.sync_copy(deg_sh.at[pl.ds(s * RPT, RPT)], zero_v)
    pltpu.sync_copy(zero_v, deg_hbm.at[pl.ds(off_n, RPT)])
    pltpu.sync_copy(ex_v, ex_hbm.at[pl.ds(base_e, EPT)])


_sc_pass1 = pl.kernel(
    _p1_body,
    out_type=[
        jax.ShapeDtypeStruct((NC * N_PAD,), jnp.float32),
        jax.ShapeDtypeStruct((NC * N_PAD,), jnp.float32),
        jax.ShapeDtypeStruct((E_PAD,), jnp.float32),
    ],
    mesh=_MESH,
    compiler_params=pltpu.CompilerParams(needs_layout_passes=False),
    scratch_types=[
        pltpu.VMEM((N_PAD,), jnp.float32),
        pltpu.VMEM((N_PAD,), jnp.float32),
        pltpu.VMEM((EPT,), jnp.int32),
        pltpu.VMEM((EPT,), jnp.int32),
        pltpu.VMEM((EPT,), jnp.float32),
        pltpu.VMEM((EB,), jnp.int32),
        pltpu.VMEM((EB,), jnp.float32),
        pltpu.VMEM((RPT,), jnp.float32),
        pltpu.VMEM((EB,), jnp.float32),
        pltpu.VMEM_SHARED((N_PAD,), jnp.float32),
        pltpu.VMEM_SHARED((N_PAD,), jnp.float32),
    ],
)


def _tc_den_body(d_ref, o_ref):
    o_ref[...] = d_ref[0] + d_ref[1]


def _tc_den(denom2r):
    return pl.pallas_call(
        _tc_den_body,
        out_shape=jax.ShapeDtypeStruct((NB, EB), jnp.float32),
    )(denom2r)


def _p2_body(h1_hbm, ex_hbm, src_hbm, dst_hbm, den_hbm,
             wsum_hbm,
             sb_v, db_v, al_v, den_v, rows_v,
             wsum_sh, sem):
    c = lax.axis_index("c")
    s = lax.axis_index("s")
    wid = c * NS + s
    base_e = wid * EPT

    pltpu.sync_copy(den_hbm, den_v)

    def _zloop(i, carry):
        rows_v[i // (D // L), pl.ds((i % (D // L)) * L, L)] = jnp.zeros((L,), jnp.float32)
        return carry
    lax.fori_loop(0, EB * D // L, _zloop, 0)
    for k in range(RPT // EB):
        pltpu.sync_copy(rows_v, wsum_sh.at[pl.ds(s * RPT + k * EB, EB)])
    rem = RPT % EB
    if rem:
        pltpu.sync_copy(rows_v.at[pl.ds(0, rem)],
                        wsum_sh.at[pl.ds(s * RPT + (RPT // EB) * EB, rem)])
    plsc.subcore_barrier()

    def _bloop(j, carry):
        off_e = base_e + j * EB
        pltpu.sync_copy(src_hbm.at[pl.ds(off_e, EB)], sb_v)
        pltpu.sync_copy(dst_hbm.at[pl.ds(off_e, EB)], db_v)
        pltpu.sync_copy(ex_hbm.at[pl.ds(off_e, EB)], al_v)

        def _aloop(b, c2):
            sl = pl.ds(b * L, L)
            dv = plsc.load_gather(den_v, [db_v[sl]])
            al_v[sl] = al_v[sl] / jnp.maximum(dv, 1e-16)
            return c2
        lax.fori_loop(0, EB // L, _aloop, 0)

        pltpu.async_copy(h1_hbm.at[sb_v], rows_v, sem).wait()

        def _rloop(r, c2):
            av = plsc.load_gather(al_v, [lax.broadcast_in_dim(r, (L,), ())])
            for q in range(D // L):
                sl = pl.ds(q * L, L)
                rows_v[r, sl] = rows_v[r, sl] * av
            return c2
        lax.fori_loop(0, EB, _rloop, 0)

        pltpu.sync_copy(rows_v, wsum_sh.at[db_v], add=True)
        return carry
    lax.fori_loop(0, NB, _bloop, 0)
    plsc.subcore_barrier()

    off_n = s * RPT
    for k in range(-(-RPT // EB)):
        sz = min(EB, RPT - k * EB)
        pltpu.sync_copy(wsum_sh.at[pl.ds(off_n + k * EB, sz)], rows_v.at[pl.ds(0, sz)])
        pltpu.sync_copy(rows_v.at[pl.ds(0, sz)], wsum_hbm.at[c].at[pl.ds(off_n + k * EB, sz)])


_sc_pass2 = pl.kernel(
    _p2_body,
    out_type=[jax.ShapeDtypeStruct((NC, N_PAD, D), jnp.float32)],
    mesh=_MESH,
    compiler_params=pltpu.CompilerParams(needs_layout_passes=False),
    scratch_types=[
        pltpu.VMEM((EB,), jnp.int32),
        pltpu.VMEM((EB,), jnp.int32),
        pltpu.VMEM((EB,), jnp.float32),
        pltpu.VMEM((N_PAD,), jnp.float32),
        pltpu.VMEM((EB, D), jnp.float32),
        pltpu.VMEM_SHARED((N_PAD, D), jnp.float32),
        pltpu.SemaphoreType.DMA,
    ],
)


def _tc_final_body(z_ref, w_ref, g_ref, o_ref):
    deg = g_ref[0, :N, :] + g_ref[1, :N, :]
    n = jnp.maximum(deg, 1.0)
    w = w_ref[0, :N, :] + w_ref[1, :N, :]
    o_ref[...] = (deg / (n * n)) * z_ref[...] + w


def _tc_final(z, wsum2, deg2):
    return pl.pallas_call(
        _tc_final_body,
        out_shape=jax.ShapeDtypeStruct((N, D), jnp.float32),
    )(z, wsum2, deg2)


def kernel(h0, h1, edge_index, W_fcdst, W_attn):
    src = edge_index[0].astype(jnp.int32)
    dst = edge_index[1].astype(jnp.int32)
    pad_e = E_PAD - E
    src1 = jnp.concatenate([src, jnp.zeros((pad_e,), jnp.int32)])
    dst1 = jnp.concatenate([dst, jnp.full((pad_e,), N, jnp.int32)])

    wt = W_fcdst.T
    wa_s = W_attn[0, :D].reshape(D, 1)
    wa_d = W_attn[0, D:].reshape(D, 1)

    z, ss, sd = _tc_prep(h0, h1, wt, wa_s, wa_d)
    zpad = jnp.zeros((N_PAD - N,), jnp.float32)
    ssp = jnp.concatenate([ss[:, 0], zpad])
    sdp = jnp.concatenate([sd[:, 0], zpad])

    denom2, deg2, ex1 = _sc_pass1(ssp, sdp, src1, dst1)
    den = _tc_den(denom2.reshape(NC, NB, EB)).reshape(N_PAD)
    (wsum2,) = _sc_pass2(h1, ex1, src1, dst1, den)
    return _tc_final(z, wsum2, deg2.reshape(NC, N_PAD, 1))


# speedup vs baseline: 12.9671x; 1.1134x over previous
import jax
import jax.numpy as jnp
from jax import lax
from jax.experimental import pallas as pl
from jax.experimental.pallas import tpu as pltpu
from jax.experimental.pallas import tpu_sc as plsc

N = 10000
D = 128
E = 320000
NC, NS, L = 2, 16, 16
NW = NC * NS
EB = 128
NB = 80
EPT = NB * EB
E_PAD = NW * EPT
N_PAD = 10112
NBN = N_PAD // EB
RPT = N_PAD // NS

_MESH = plsc.VectorSubcoreMesh(
    core_axis_name="c", subcore_axis_name="s", num_cores=NC, num_subcores=NS
)


def _tc_prep_body(h0_ref, h1_ref, wt_ref, was_ref, wad_ref, z_ref, ss_ref, sd_ref):
    z = jnp.dot(h0_ref[...], wt_ref[...], preferred_element_type=jnp.float32)
    z_ref[...] = z
    ss_ref[...] = jnp.dot(h1_ref[...], was_ref[...], preferred_element_type=jnp.float32)
    sd_ref[...] = jnp.dot(z, wad_ref[...], preferred_element_type=jnp.float32)


def _tc_prep(h0, h1, wt, wa_s, wa_d):
    return pl.pallas_call(
        _tc_prep_body,
        out_shape=(
            jax.ShapeDtypeStruct((N, D), jnp.float32),
            jax.ShapeDtypeStruct((N, 1), jnp.float32),
            jax.ShapeDtypeStruct((N, 1), jnp.float32),
        ),
    )(h0, h1, wt, wa_s, wa_d)


def _xlane_max(v, buf):
    for sh in (8, 4, 2, 1):
        buf[pl.ds(0, L)] = v
        idx = lax.broadcasted_iota(jnp.int32, (L,), 0) ^ sh
        v = jnp.maximum(v, plsc.load_gather(buf, [idx]))
    return v


def _p1_body(ss_hbm, sd_hbm, src_hbm, dst_hbm,
             denom_hbm, deg_hbm, ex_hbm,
             ssrc_v, sdst_v, src_v, dst_v, ex_v, db_v, ones_v, zero_v, mred_v,
             denom_sh, deg_sh):
    c = lax.axis_index("c")
    s = lax.axis_index("s")
    wid = c * NS + s
    base_e = wid * EPT

    pltpu.sync_copy(ss_hbm, ssrc_v)
    pltpu.sync_copy(sd_hbm, sdst_v)
    pltpu.sync_copy(src_hbm.at[pl.ds(base_e, EPT)], src_v)
    pltpu.sync_copy(dst_hbm.at[pl.ds(base_e, EPT)], dst_v)

    def _zloop(i, carry):
        zero_v[pl.ds(i * L, L)] = jnp.zeros((L,), jnp.float32)
        return carry
    lax.fori_loop(0, -(-RPT // L), _zloop, 0)
    for i in range(EB // L):
        ones_v[pl.ds(i * L, L)] = jnp.ones((L,), jnp.float32)

    pltpu.sync_copy(zero_v.at[pl.ds(0, RPT)], denom_sh.at[pl.ds(s * RPT, RPT)])
    pltpu.sync_copy(zero_v.at[pl.ds(0, RPT)], deg_sh.at[pl.ds(s * RPT, RPT)])
    plsc.subcore_barrier()

    def _mloop(i, carry):
        ms, md = carry
        sl = pl.ds(i * L, L)
        return jnp.maximum(ms, ssrc_v[sl]), jnp.maximum(md, sdst_v[sl])
    init = jnp.full((L,), -3e38, jnp.float32)
    ms, md = lax.fori_loop(0, N_PAD // L, _mloop, (init, init))
    m = _xlane_max(ms, mred_v) + _xlane_max(md, mred_v)
    mv = jnp.where(m > 0, m, m * 0.01)

    def _eloop(b, carry):
        sl = pl.ds(b * L, L)
        vs = plsc.load_gather(ssrc_v, [src_v[sl]])
        vd = plsc.load_gather(sdst_v, [dst_v[sl]])
        a = vs + vd
        e = jnp.where(a > 0, a, a * 0.01)
        ex_v[sl] = jnp.exp(e - mv)
        return carry
    lax.fori_loop(0, EPT // L, _eloop, 0)

    def _sloop(j, carry):
        for i in range(EB // L):
            db_v[pl.ds(i * L, L)] = dst_v[pl.ds(j * EB + i * L, L)]
        pltpu.sync_copy(ex_v.at[pl.ds(j * EB, EB)], denom_sh.at[db_v], add=True)
        pltpu.sync_copy(ones_v, deg_sh.at[db_v], add=True)
        return carry
    lax.fori_loop(0, NB, _sloop, 0)
    plsc.subcore_barrier()

    off_n = c * N_PAD + s * RPT
    pltpu.sync_copy(denom_sh.at[pl.ds(s * RPT, RPT)], zero_v.at[pl.ds(0, RPT)])
    pltpu.sync_copy(zero_v.at[pl.ds(0, RPT)], denom_hbm.at[pl.ds(off_n, RPT)])
    pltpu.sync_copy(deg_sh.at[pl.ds(s * RPT, RPT)], zero_v.at[pl.ds(0, RPT)])
    pltpu.sync_copy(zero_v.at[pl.ds(0, RPT)], deg_hbm.at[pl.ds(off_n, RPT)])
    pltpu.sync_copy(ex_v, ex_hbm.at[pl.ds(base_e, EPT)])


_sc_pass1 = pl.kernel(
    _p1_body,
    out_type=[
        jax.ShapeDtypeStruct((NC * N_PAD,), jnp.float32),
        jax.ShapeDtypeStruct((NC * N_PAD,), jnp.float32),
        jax.ShapeDtypeStruct((E_PAD,), jnp.float32),
    ],
    mesh=_MESH,
    compiler_params=pltpu.CompilerParams(needs_layout_passes=False),
    scratch_types=[
        pltpu.VMEM((N_PAD,), jnp.float32),
        pltpu.VMEM((N_PAD,), jnp.float32),
        pltpu.VMEM((EPT,), jnp.int32),
        pltpu.VMEM((EPT,), jnp.int32),
        pltpu.VMEM((EPT,), jnp.float32),
        pltpu.VMEM((EB,), jnp.int32),
        pltpu.VMEM((EB,), jnp.float32),
        pltpu.VMEM((-(-RPT // L) * L,), jnp.float32),
        pltpu.VMEM((EB,), jnp.float32),
        pltpu.VMEM_SHARED((N_PAD,), jnp.float32),
        pltpu.VMEM_SHARED((N_PAD,), jnp.float32),
    ],
)


def _tc_den_body(d_ref, o_ref):
    o_ref[...] = d_ref[0] + d_ref[1]


def _tc_den(denom2r):
    return pl.pallas_call(
        _tc_den_body,
        out_shape=jax.ShapeDtypeStruct((NBN, EB), jnp.float32),
    )(denom2r)


def _p2_body(h1_hbm, ex_hbm, src_hbm, dst_hbm, den_hbm,
             wsum_hbm,
             sbA, sbB, dbA, dbB, alA, alB, den_v, rowsA, rowsB,
             wsum_sh, semIA, semIB, semDA, semDB, semXA, semXB, semGA, semGB):
    c = lax.axis_index("c")
    s = lax.axis_index("s")
    wid = c * NS + s
    base_e = wid * EPT

    pltpu.sync_copy(den_hbm, den_v)

    def _zloop(i, carry):
        rowsA[i // (D // L), pl.ds((i % (D // L)) * L, L)] = jnp.zeros((L,), jnp.float32)
        return carry
    lax.fori_loop(0, EB * D // L, _zloop, 0)
    for k in range(-(-RPT // EB)):
        sz = min(EB, RPT - k * EB)
        pltpu.sync_copy(rowsA.at[pl.ds(0, sz)],
                        wsum_sh.at[pl.ds(s * RPT + k * EB, sz)])
    plsc.subcore_barrier()

    def _fetch_idx(b, sb, db, al, semI, semD, semX):
        off_e = base_e + b * EB
        pltpu.async_copy(src_hbm.at[pl.ds(off_e, EB)], sb, semI)
        pltpu.async_copy(dst_hbm.at[pl.ds(off_e, EB)], db, semD)
        pltpu.async_copy(ex_hbm.at[pl.ds(off_e, EB)], al, semX)

    def _wait_idx(b, sb, db, al, semI, semD, semX):
        off_e = base_e + b * EB
        pltpu.make_async_copy(src_hbm.at[pl.ds(off_e, EB)], sb, semI).wait()
        pltpu.make_async_copy(dst_hbm.at[pl.ds(off_e, EB)], db, semD).wait()
        pltpu.make_async_copy(ex_hbm.at[pl.ds(off_e, EB)], al, semX).wait()

    def _compute(b, sb, db, al, rows, semG):
        pltpu.make_async_copy(h1_hbm.at[sb], rows, semG).wait()
        for i in range(EB // L):
            sl = pl.ds(i * L, L)
            dv = plsc.load_gather(den_v, [db[sl]])
            al[sl] = al[sl] / jnp.maximum(dv, 1e-16)
        def _rloop(g, c2):
            bi = lax.broadcast_in_dim(g * 4, (L,), ())
            for u in range(4):
                av = plsc.load_gather(al, [bi + u])
                r = g * 4 + u
                for q in range(D // L):
                    sl = pl.ds(q * L, L)
                    rows[r, sl] = rows[r, sl] * av
            return c2
        lax.fori_loop(0, EB // 4, _rloop, 0)
        pltpu.sync_copy(rows, wsum_sh.at[db], add=True)

    _fetch_idx(0, sbA, dbA, alA, semIA, semDA, semXA)
    _wait_idx(0, sbA, dbA, alA, semIA, semDA, semXA)
    pltpu.async_copy(h1_hbm.at[sbA], rowsA, semGA)
    _fetch_idx(1, sbB, dbB, alB, semIB, semDB, semXB)

    def _ploop(k, carry):
        b = 2 * k
        _wait_idx(b + 1, sbB, dbB, alB, semIB, semDB, semXB)
        pltpu.async_copy(h1_hbm.at[sbB], rowsB, semGB)
        _compute(b, sbA, dbA, alA, rowsA, semGA)

        @pl.when(b + 2 < NB)
        def _():
            _fetch_idx(b + 2, sbA, dbA, alA, semIA, semDA, semXA)

        @pl.when(b + 2 < NB)
        def _():
            _wait_idx(b + 2, sbA, dbA, alA, semIA, semDA, semXA)
            pltpu.async_copy(h1_hbm.at[sbA], rowsA, semGA)
        _compute(b + 1, sbB, dbB, alB, rowsB, semGB)

        @pl.when(b + 3 < NB)
        def _():
            _fetch_idx(b + 3, sbB, dbB, alB, semIB, semDB, semXB)
        return carry
    lax.fori_loop(0, NB // 2, _ploop, 0)
    plsc.subcore_barrier()

    off_n = s * RPT
    for k in range(-(-RPT // EB)):
        sz = min(EB, RPT - k * EB)
        pltpu.sync_copy(wsum_sh.at[pl.ds(off_n + k * EB, sz)], rowsA.at[pl.ds(0, sz)])
        pltpu.sync_copy(rowsA.at[pl.ds(0, sz)], wsum_hbm.at[c].at[pl.ds(off_n + k * EB, sz)])


_sc_pass2 = pl.kernel(
    _p2_body,
    out_type=[jax.ShapeDtypeStruct((NC, N_PAD, D), jnp.float32)],
    mesh=_MESH,
    compiler_params=pltpu.CompilerParams(needs_layout_passes=False),
    scratch_types=[
        pltpu.VMEM((EB,), jnp.int32),
        pltpu.VMEM((EB,), jnp.int32),
        pltpu.VMEM((EB,), jnp.int32),
        pltpu.VMEM((EB,), jnp.int32),
        pltpu.VMEM((EB,), jnp.float32),
        pltpu.VMEM((EB,), jnp.float32),
        pltpu.VMEM((N_PAD,), jnp.float32),
        pltpu.VMEM((EB, D), jnp.float32),
        pltpu.VMEM((EB, D), jnp.float32),
        pltpu.VMEM_SHARED((N_PAD, D), jnp.float32),
        pltpu.SemaphoreType.DMA,
        pltpu.SemaphoreType.DMA,
        pltpu.SemaphoreType.DMA,
        pltpu.SemaphoreType.DMA,
        pltpu.SemaphoreType.DMA,
        pltpu.SemaphoreType.DMA,
        pltpu.SemaphoreType.DMA,
        pltpu.SemaphoreType.DMA,
    ],
)


def _tc_final_body(z_ref, w_ref, g_ref, o_ref):
    deg = g_ref[0, :N, :] + g_ref[1, :N, :]
    n = jnp.maximum(deg, 1.0)
    w = w_ref[0, :N, :] + w_ref[1, :N, :]
    o_ref[...] = (deg / (n * n)) * z_ref[...] + w


def _tc_final(z, wsum2, deg2):
    return pl.pallas_call(
        _tc_final_body,
        out_shape=jax.ShapeDtypeStruct((N, D), jnp.float32),
    )(z, wsum2, deg2)


def kernel(h0, h1, edge_index, W_fcdst, W_attn):
    src = edge_index[0].astype(jnp.int32)
    dst = edge_index[1].astype(jnp.int32)
    pad_e = E_PAD - E
    src1 = jnp.concatenate([src, jnp.zeros((pad_e,), jnp.int32)])
    dst1 = jnp.concatenate([dst, jnp.full((pad_e,), N, jnp.int32)])

    wt = W_fcdst.T
    wa_s = W_attn[0, :D].reshape(D, 1)
    wa_d = W_attn[0, D:].reshape(D, 1)

    z, ss, sd = _tc_prep(h0, h1, wt, wa_s, wa_d)
    zpad = jnp.zeros((N_PAD - N,), jnp.float32)
    ssp = jnp.concatenate([ss[:, 0], zpad])
    sdp = jnp.concatenate([sd[:, 0], zpad])

    denom2, deg2, ex1 = _sc_pass1(ssp, sdp, src1, dst1)
    den = _tc_den(denom2.reshape(NC, NBN, EB)).reshape(N_PAD)
    (wsum2,) = _sc_pass2(h1, ex1, src1, dst1, den)
    return _tc_final(z, wsum2, deg2.reshape(NC, N_PAD, 1))
